# Initial kernel scaffold; baseline (speedup 1.0000x reference)
import jax
import jax.numpy as jnp
from jax.experimental import pallas as pl


def kernel(x, conv_layer__w, conv_layer__gamma, conv_layer__beta, conv_layer__mean, conv_layer__var, encoder1__w, encoder1__gamma, encoder1__beta, encoder1__mean, encoder1__var, encoder2__w, encoder2__gamma, encoder2__beta, encoder2__mean, encoder2__var, encoder3__w, encoder3__gamma, encoder3__beta, encoder3__mean, encoder3__var, post_transformer__w, post_transformer__gamma, post_transformer__beta, post_transformer__mean, post_transformer__var, post_transformer__b, vit__proj_w, vit__proj_b, vit__cls, vit__pos, vit_block0__wq, vit_block0__wk, vit_block0__wv, vit_block0__wo, vit_block0__ln1_g, vit_block0__ln1_b, vit_block0__w1, vit_block0__b1, vit_block0__w2, vit_block0__b2, vit_block0__ln2_g, vit_block0__ln2_b, vit_block1__wq, vit_block1__wk, vit_block1__wv, vit_block1__wo, vit_block1__ln1_g, vit_block1__ln1_b, vit_block1__w1, vit_block1__b1, vit_block1__w2, vit_block1__b2, vit_block1__ln2_g, vit_block1__ln2_b, vit_block2__wq, vit_block2__wk, vit_block2__wv, vit_block2__wo, vit_block2__ln1_g, vit_block2__ln1_b, vit_block2__w1, vit_block2__b1, vit_block2__w2, vit_block2__b2, vit_block2__ln2_g, vit_block2__ln2_b, vit_block3__wq, vit_block3__wk, vit_block3__wv, vit_block3__wo, vit_block3__ln1_g, vit_block3__ln1_b, vit_block3__w1, vit_block3__b1, vit_block3__w2, vit_block3__b2, vit_block3__ln2_g, vit_block3__ln2_b):
    raise NotImplementedError("write your pallas kernel here")


# speedup vs baseline: 2.4160x; 2.4160x over previous
import math

import jax
import jax.numpy as jnp
from jax.experimental import pallas as pl
from jax.experimental.pallas import tpu as pltpu

_BN_EPS = 1e-5
_LN_EPS = 1e-5
_ROWS = 256


def _gemm_kernel(M):
    def body(p_ref, w_ref, s_ref, b_ref, o_ref):
        for s0 in range(0, M, _ROWS):
            ch = min(_ROWS, M - s0)
            acc = jnp.dot(p_ref[s0:s0 + ch, :], w_ref[...],
                          preferred_element_type=jnp.float32)
            y = acc * s_ref[...] + b_ref[...]
            o_ref[s0:s0 + ch, :] = jnp.maximum(y, 0.0)
    return body


def _conv_gemm(patches, w2d, scale, bias):
    N, M, K = patches.shape
    C = w2d.shape[1]
    cost = pl.CostEstimate(
        flops=2 * N * M * K * C, transcendentals=0,
        bytes_accessed=2 * N * M * K + 2 * K * C + 4 * N * M * C)
    return pl.pallas_call(
        _gemm_kernel(M),
        out_shape=jax.ShapeDtypeStruct((N, M, C), jnp.float32),
        grid=(N,),
        in_specs=[
            pl.BlockSpec((pl.Squeezed(), M, K), lambda n: (n, 0, 0)),
            pl.BlockSpec((K, C), lambda n: (0, 0)),
            pl.BlockSpec((1, C), lambda n: (0, 0)),
            pl.BlockSpec((1, C), lambda n: (0, 0)),
        ],
        out_specs=pl.BlockSpec((pl.Squeezed(), M, C), lambda n: (n, 0, 0)),
        compiler_params=pltpu.CompilerParams(
            dimension_semantics=("parallel",)),
        cost_estimate=cost,
    )(patches, w2d, scale[None, :], bias[None, :])


def _fold_bn(gamma, beta, mean, var, b=None):
    scale = gamma * jax.lax.rsqrt(var + _BN_EPS)
    bias = beta - mean * scale
    if b is not None:
        bias = bias + b * scale
    return scale, bias


def _windows(xp, n_out, win, stride):
    N = xp.shape[0]
    C = xp.shape[3]
    xp = xp.astype(jnp.bfloat16)
    cols = []
    for wy in range(win):
        for wx in range(win):
            cols.append(xp[:, wy::stride, wx::stride, :][:, :n_out, :n_out, :])
    pat = jnp.concatenate([c[..., None, :] for c in cols], axis=3)
    return pat.reshape(N, n_out * n_out, win * win * C)


def _packed_weight(w, win):
    k, _, cin, cout = w.shape
    wp = jnp.zeros((win, win, cin, 2, 2, cout), jnp.float32)
    for sy in range(2):
        for sx in range(2):
            wp = wp.at[2 * sy:2 * sy + k, 2 * sx:2 * sx + k, :, sy, sx, :].set(w)
    return wp.reshape(win * win * cin, 4 * cout).astype(jnp.bfloat16)


def _unpack_phases(y, n_out, cout):
    N = y.shape[0]
    y = y.reshape(N, n_out, n_out, 2, 2, cout)
    y = jnp.transpose(y, (0, 1, 3, 2, 4, 5))
    return y.reshape(N, 2 * n_out, 2 * n_out, cout)


def _conv_s2_packed(x, w, gamma, beta, mean, var, k, pad):
    N, H, _, cin = x.shape
    cout = w.shape[3]
    win = k + 2
    ho = H // 2
    xp = jnp.pad(x, ((0, 0), (pad, pad), (pad, pad), (0, 0)))
    pat = _windows(xp, ho // 2, win, 4)
    w2d = _packed_weight(w, win)
    scale, bias = _fold_bn(gamma, beta, mean, var)
    y = _conv_gemm(pat, w2d, jnp.tile(scale, 4), jnp.tile(bias, 4))
    return _unpack_phases(y, ho // 2, cout)


def _conv_plain(x, w, gamma, beta, mean, var, stride, b=None):
    N, H, _, cin = x.shape
    k = w.shape[0]
    cout = w.shape[3]
    pad = (k - 1) // 2
    ho = (H + 2 * pad - k) // stride + 1
    xp = jnp.pad(x, ((0, 0), (pad, pad), (pad, pad), (0, 0)))
    pat = _windows(xp, ho, k, stride)
    w2d = w.reshape(k * k * cin, cout).astype(jnp.bfloat16)
    scale, bias = _fold_bn(gamma, beta, mean, var, b)
    y = _conv_gemm(pat, w2d, scale, bias)
    return y.reshape(N, ho, ho, cout)


def _vit_kernel(num_blocks, num_heads, TP, T0, D, dh, att_scale):
    def body(tok_ref, pw_ref, pb_ref, pre_ref, sel_ref, msk_ref,
             wqkv_ref, wo_ref, w1_ref, b1_ref, w2_ref, b2_ref,
             l1g_ref, l1b_ref, l2g_ref, l2b_ref, o_ref):
        f32 = jnp.float32
        bf = jnp.bfloat16

        def ln(v, g, b):
            mu = jnp.mean(v, axis=-1, keepdims=True)
            vc = v - mu
            var = jnp.mean(vc * vc, axis=-1, keepdims=True)
            return vc * jax.lax.rsqrt(var + _LN_EPS) * g + b

        def tanh(z):
            return 1.0 - 2.0 / (jnp.exp(2.0 * z) + 1.0)

        proj = jnp.dot(tok_ref[...], pw_ref[...],
                       preferred_element_type=f32) + pb_ref[...]
        x = pre_ref[...] + jnp.dot(sel_ref[...], proj.astype(bf),
                                   preferred_element_type=f32)
        msk = msk_ref[...]

        for blk in range(num_blocks):
            xb = x.astype(bf)
            qkv = jnp.dot(xb, wqkv_ref[blk],
                          preferred_element_type=f32)
            outs = []
            for h in range(num_heads):
                qh = qkv[:, h * dh:(h + 1) * dh].astype(bf)
                kh = qkv[:, D + h * dh:D + (h + 1) * dh].astype(bf)
                vh = qkv[:, 2 * D + h * dh:2 * D + (h + 1) * dh].astype(bf)
                s = jax.lax.dot_general(qh, kh, (((1,), (1,)), ((), ())),
                                        preferred_element_type=f32)
                s = s * att_scale + msk
                s = s - jnp.max(s, axis=-1, keepdims=True)
                e = jnp.exp(s)
                p = e / jnp.sum(e, axis=-1, keepdims=True)
                outs.append(jnp.dot(p.astype(bf), vh,
                                    preferred_element_type=f32))
            cat = jnp.concatenate(outs, axis=1).astype(bf)
            att = jnp.dot(cat, wo_ref[blk], preferred_element_type=f32)
            x = ln(x + att, l1g_ref[blk], l1b_ref[blk])
            m = jnp.dot(x.astype(bf), w1_ref[blk],
                        preferred_element_type=f32) + b1_ref[blk]
            m = 0.5 * m * (1.0 + tanh(0.7978845608028654 *
                                      (m + 0.044715 * m * m * m)))
            m = jnp.dot(m.astype(bf), w2_ref[blk],
                        preferred_element_type=f32) + b2_ref[blk]
            x = ln(x + m, l2g_ref[blk], l2b_ref[blk])

        o_ref[...] = x[1:T0 + 1, :]

    return body


def _vit(tokens_bf, proj_w, proj_b, cls, pos, blocks):
    N, T0, D = tokens_bf.shape
    B = len(blocks)
    heads = blocks[0]["wq"].shape[0]
    dh = blocks[0]["wq"].shape[2]
    mlp = blocks[0]["w1"].shape[1]
    TP = ((T0 + 1 + 7) // 8) * 8

    bf = jnp.bfloat16
    pre = jnp.zeros((TP, D), jnp.float32)
    pre = pre.at[:T0 + 1].set(
        jnp.concatenate([cls, jnp.zeros((T0, D), jnp.float32)], axis=0) + pos)
    sel = jnp.zeros((TP, T0), jnp.float32)
    sel = sel.at[1 + jnp.arange(T0), jnp.arange(T0)].set(1.0)
    msk = jnp.where(jnp.arange(TP)[None, :] < T0 + 1, 0.0, -1e9
                    ).astype(jnp.float32)

    def cath(name):
        return jnp.stack(
            [jnp.transpose(b[name], (1, 0, 2)).reshape(D, heads * dh)
             for b in blocks], axis=0)

    wqkv = jnp.concatenate([cath("wq"), cath("wk"), cath("wv")],
                           axis=2).astype(bf)
    wo = jnp.stack([b["wo"].reshape(heads * dh, D) for b in blocks],
                   axis=0).astype(bf)
    w1 = jnp.stack([b["w1"] for b in blocks], 0).astype(bf)
    w2 = jnp.stack([b["w2"] for b in blocks], 0).astype(bf)
    stk1 = lambda n: jnp.stack([b[n] for b in blocks], 0)[:, None, :]
    b1, b2 = stk1("b1"), stk1("b2")
    l1g, l1b, l2g, l2b = (stk1("ln1_g"), stk1("ln1_b"),
                          stk1("ln2_g"), stk1("ln2_b"))

    weights = [proj_w.astype(bf), proj_b[None, :], pre, sel.astype(bf), msk,
               wqkv, wo, w1, b1, w2, b2, l1g, l1b, l2g, l2b]
    in_specs = [pl.BlockSpec((pl.Squeezed(), T0, D), lambda n: (n, 0, 0))]
    for wgt in weights:
        in_specs.append(
            pl.BlockSpec(wgt.shape, lambda n, _nd=wgt.ndim: (0,) * _nd))

    cost = pl.CostEstimate(
        flops=N * B * (8 * TP * D * D + 4 * heads * TP * TP * dh
                       + 4 * TP * D * mlp),
        transcendentals=N * B * (heads * TP * TP + TP * mlp),
        bytes_accessed=2 * N * T0 * D + 4 * N * T0 * D
        + sum(int(w.size) * w.dtype.itemsize for w in weights))

    return pl.pallas_call(
        _vit_kernel(B, heads, TP, T0, D, dh, 1.0 / math.sqrt(dh)),
        out_shape=jax.ShapeDtypeStruct((N, T0, D), jnp.float32),
        grid=(N,),
        in_specs=in_specs,
        out_specs=pl.BlockSpec((pl.Squeezed(), T0, D), lambda n: (n, 0, 0)),
        compiler_params=pltpu.CompilerParams(
            dimension_semantics=("parallel",)),
        cost_estimate=cost,
    )(tokens_bf, *weights)


def kernel(x, conv_layer__w, conv_layer__gamma, conv_layer__beta, conv_layer__mean, conv_layer__var, encoder1__w, encoder1__gamma, encoder1__beta, encoder1__mean, encoder1__var, encoder2__w, encoder2__gamma, encoder2__beta, encoder2__mean, encoder2__var, encoder3__w, encoder3__gamma, encoder3__beta, encoder3__mean, encoder3__var, post_transformer__w, post_transformer__gamma, post_transformer__beta, post_transformer__mean, post_transformer__var, post_transformer__b, vit__proj_w, vit__proj_b, vit__cls, vit__pos, vit_block0__wq, vit_block0__wk, vit_block0__wv, vit_block0__wo, vit_block0__ln1_g, vit_block0__ln1_b, vit_block0__w1, vit_block0__b1, vit_block0__w2, vit_block0__b2, vit_block0__ln2_g, vit_block0__ln2_b, vit_block1__wq, vit_block1__wk, vit_block1__wv, vit_block1__wo, vit_block1__ln1_g, vit_block1__ln1_b, vit_block1__w1, vit_block1__b1, vit_block1__w2, vit_block1__b2, vit_block1__ln2_g, vit_block1__ln2_b, vit_block2__wq, vit_block2__wk, vit_block2__wv, vit_block2__wo, vit_block2__ln1_g, vit_block2__ln1_b, vit_block2__w1, vit_block2__b1, vit_block2__w2, vit_block2__b2, vit_block2__ln2_g, vit_block2__ln2_b, vit_block3__wq, vit_block3__wk, vit_block3__wv, vit_block3__wo, vit_block3__ln1_g, vit_block3__ln1_b, vit_block3__w1, vit_block3__b1, vit_block3__w2, vit_block3__b2, vit_block3__ln2_g, vit_block3__ln2_b):
    h = jnp.transpose(x, (0, 2, 3, 1))

    x1 = _conv_s2_packed(h, conv_layer__w, conv_layer__gamma,
                         conv_layer__beta, conv_layer__mean, conv_layer__var,
                         k=7, pad=3)
    x2 = _conv_s2_packed(x1, encoder1__w, encoder1__gamma, encoder1__beta,
                         encoder1__mean, encoder1__var, k=3, pad=1)
    x3 = _conv_s2_packed(x2, encoder2__w, encoder2__gamma, encoder2__beta,
                         encoder2__mean, encoder2__var, k=3, pad=1)
    x4 = _conv_plain(x3, encoder3__w, encoder3__gamma, encoder3__beta,
                     encoder3__mean, encoder3__var, stride=2)

    N, Hv, Wv, D = x4.shape
    tokens = x4.reshape(N, Hv * Wv, D).astype(jnp.bfloat16)

    blocks = []
    for i, pfx in enumerate([
            (vit_block0__wq, vit_block0__wk, vit_block0__wv, vit_block0__wo,
             vit_block0__ln1_g, vit_block0__ln1_b, vit_block0__w1,
             vit_block0__b1, vit_block0__w2, vit_block0__b2,
             vit_block0__ln2_g, vit_block0__ln2_b),
            (vit_block1__wq, vit_block1__wk, vit_block1__wv, vit_block1__wo,
             vit_block1__ln1_g, vit_block1__ln1_b, vit_block1__w1,
             vit_block1__b1, vit_block1__w2, vit_block1__b2,
             vit_block1__ln2_g, vit_block1__ln2_b),
            (vit_block2__wq, vit_block2__wk, vit_block2__wv, vit_block2__wo,
             vit_block2__ln1_g, vit_block2__ln1_b, vit_block2__w1,
             vit_block2__b1, vit_block2__w2, vit_block2__b2,
             vit_block2__ln2_g, vit_block2__ln2_b),
            (vit_block3__wq, vit_block3__wk, vit_block3__wv, vit_block3__wo,
             vit_block3__ln1_g, vit_block3__ln1_b, vit_block3__w1,
             vit_block3__b1, vit_block3__w2, vit_block3__b2,
             vit_block3__ln2_g, vit_block3__ln2_b)]):
        wq, wk, wv, wo, l1g, l1b, w1, b1, w2, b2, l2g, l2b = pfx
        blocks.append({"wq": wq, "wk": wk, "wv": wv, "wo": wo,
                       "ln1_g": l1g, "ln1_b": l1b, "w1": w1, "b1": b1,
                       "w2": w2, "b2": b2, "ln2_g": l2g, "ln2_b": l2b})

    tv = _vit(tokens, vit__proj_w, vit__proj_b, vit__cls, vit__pos, blocks)
    xv = tv.reshape(N, Hv, Wv, D)

    xo = _conv_plain(xv, post_transformer__w, post_transformer__gamma,
                     post_transformer__beta, post_transformer__mean,
                     post_transformer__var, stride=1, b=post_transformer__b)

    nchw = lambda t: jnp.transpose(t, (0, 3, 1, 2))
    return nchw(xo), nchw(x1), nchw(x2), nchw(x3)


# speedup vs baseline: 4.9577x; 2.0520x over previous
import math

import jax
import jax.numpy as jnp
from jax.experimental import pallas as pl
from jax.experimental.pallas import tpu as pltpu

_BN_EPS = 1e-5
_LN_EPS = 1e-5
_ROWS = 256


def _gemm_kernel(M):
    def body(p_ref, w_ref, s_ref, b_ref, o_ref):
        for s0 in range(0, M, _ROWS):
            ch = min(_ROWS, M - s0)
            acc = jnp.dot(p_ref[s0:s0 + ch, :], w_ref[...],
                          preferred_element_type=jnp.float32)
            y = acc * s_ref[...] + b_ref[...]
            o_ref[s0:s0 + ch, :] = jnp.maximum(y, 0.0)
    return body


def _conv_gemm(patches, w2d, scale, bias):
    N, M, K = patches.shape
    C = w2d.shape[1]
    cost = pl.CostEstimate(
        flops=2 * N * M * K * C, transcendentals=0,
        bytes_accessed=2 * N * M * K + 2 * K * C + 4 * N * M * C)
    return pl.pallas_call(
        _gemm_kernel(M),
        out_shape=jax.ShapeDtypeStruct((N, M, C), jnp.float32),
        grid=(N,),
        in_specs=[
            pl.BlockSpec((pl.Squeezed(), M, K), lambda n: (n, 0, 0)),
            pl.BlockSpec((K, C), lambda n: (0, 0)),
            pl.BlockSpec((1, C), lambda n: (0, 0)),
            pl.BlockSpec((1, C), lambda n: (0, 0)),
        ],
        out_specs=pl.BlockSpec((pl.Squeezed(), M, C), lambda n: (n, 0, 0)),
        compiler_params=pltpu.CompilerParams(
            dimension_semantics=("parallel",)),
        cost_estimate=cost,
    )(patches, w2d, scale[None, :], bias[None, :])


def _fold_bn(gamma, beta, mean, var, b=None):
    scale = gamma * jax.lax.rsqrt(var + _BN_EPS)
    bias = beta - mean * scale
    if b is not None:
        bias = bias + b * scale
    return scale, bias


def _windows(x, n_out, win, stride, pad):
    N = x.shape[0]
    C = x.shape[3]
    H = x.shape[1]
    hi = stride * (n_out - 1) + win - H - pad
    pat = jax.lax.conv_general_dilated_patches(
        x.astype(jnp.bfloat16), (win, win), (stride, stride),
        [(pad, max(hi, 0)), (pad, max(hi, 0))],
        dimension_numbers=("NHWC", "HWIO", "NHWC"))
    return pat.reshape(N, n_out * n_out, C * win * win)


def _packed_weight(w, win):
    k, _, cin, cout = w.shape
    wp = jnp.zeros((win, win, cin, 2, 2, cout), jnp.float32)
    for sy in range(2):
        for sx in range(2):
            wp = wp.at[2 * sy:2 * sy + k, 2 * sx:2 * sx + k, :, sy, sx, :].set(w)
    wp = jnp.transpose(wp, (2, 0, 1, 3, 4, 5))
    return wp.reshape(cin * win * win, 4 * cout).astype(jnp.bfloat16)


def _unpack_phases(y, n_out, cout):
    N = y.shape[0]
    y = y.reshape(N, n_out, n_out, 2, 2, cout)
    y = jnp.transpose(y, (0, 1, 3, 2, 4, 5))
    return y.reshape(N, 2 * n_out, 2 * n_out, cout)


def _conv_s2_packed(x, w, gamma, beta, mean, var, k, pad):
    N, H, _, cin = x.shape
    cout = w.shape[3]
    win = k + 2
    ho = H // 2
    pat = _windows(x, ho // 2, win, 4, pad)
    w2d = _packed_weight(w, win)
    scale, bias = _fold_bn(gamma, beta, mean, var)
    y = _conv_gemm(pat, w2d, jnp.tile(scale, 4), jnp.tile(bias, 4))
    return _unpack_phases(y, ho // 2, cout)


def _conv_plain(x, w, gamma, beta, mean, var, stride, b=None):
    N, H, _, cin = x.shape
    k = w.shape[0]
    cout = w.shape[3]
    pad = (k - 1) // 2
    ho = (H + 2 * pad - k) // stride + 1
    pat = _windows(x, ho, k, stride, pad)
    w2d = jnp.transpose(w, (2, 0, 1, 3)).reshape(
        k * k * cin, cout).astype(jnp.bfloat16)
    scale, bias = _fold_bn(gamma, beta, mean, var, b)
    y = _conv_gemm(pat, w2d, scale, bias)
    return y.reshape(N, ho, ho, cout)


def _vit_kernel(num_blocks, num_heads, TP, T0, D, dh, att_scale):
    def body(tok_ref, pw_ref, pb_ref, pre_ref, sel_ref, msk_ref,
             wqkv_ref, wo_ref, w1_ref, b1_ref, w2_ref, b2_ref,
             l1g_ref, l1b_ref, l2g_ref, l2b_ref, o_ref):
        f32 = jnp.float32
        bf = jnp.bfloat16

        def ln(v, g, b):
            mu = jnp.mean(v, axis=-1, keepdims=True)
            vc = v - mu
            var = jnp.mean(vc * vc, axis=-1, keepdims=True)
            return vc * jax.lax.rsqrt(var + _LN_EPS) * g + b

        def tanh(z):
            return 1.0 - 2.0 / (jnp.exp(2.0 * z) + 1.0)

        proj = jnp.dot(tok_ref[...], pw_ref[...],
                       preferred_element_type=f32) + pb_ref[...]
        x = pre_ref[...] + jnp.dot(sel_ref[...], proj.astype(bf),
                                   preferred_element_type=f32)
        msk = msk_ref[...]

        for blk in range(num_blocks):
            xb = x.astype(bf)
            qkv = jnp.dot(xb, wqkv_ref[blk],
                          preferred_element_type=f32)
            outs = []
            for h in range(num_heads):
                qh = qkv[:, h * dh:(h + 1) * dh].astype(bf)
                kh = qkv[:, D + h * dh:D + (h + 1) * dh].astype(bf)
                vh = qkv[:, 2 * D + h * dh:2 * D + (h + 1) * dh].astype(bf)
                s = jax.lax.dot_general(qh, kh, (((1,), (1,)), ((), ())),
                                        preferred_element_type=f32)
                s = s * att_scale + msk
                s = s - jnp.max(s, axis=-1, keepdims=True)
                e = jnp.exp(s)
                p = e / jnp.sum(e, axis=-1, keepdims=True)
                outs.append(jnp.dot(p.astype(bf), vh,
                                    preferred_element_type=f32))
            cat = jnp.concatenate(outs, axis=1).astype(bf)
            att = jnp.dot(cat, wo_ref[blk], preferred_element_type=f32)
            x = ln(x + att, l1g_ref[blk], l1b_ref[blk])
            m = jnp.dot(x.astype(bf), w1_ref[blk],
                        preferred_element_type=f32) + b1_ref[blk]
            m = 0.5 * m * (1.0 + tanh(0.7978845608028654 *
                                      (m + 0.044715 * m * m * m)))
            m = jnp.dot(m.astype(bf), w2_ref[blk],
                        preferred_element_type=f32) + b2_ref[blk]
            x = ln(x + m, l2g_ref[blk], l2b_ref[blk])

        o_ref[...] = x[1:T0 + 1, :]

    return body


def _vit(tokens_bf, proj_w, proj_b, cls, pos, blocks):
    N, T0, D = tokens_bf.shape
    B = len(blocks)
    heads = blocks[0]["wq"].shape[0]
    dh = blocks[0]["wq"].shape[2]
    mlp = blocks[0]["w1"].shape[1]
    TP = ((T0 + 1 + 7) // 8) * 8

    bf = jnp.bfloat16
    pre = jnp.zeros((TP, D), jnp.float32)
    pre = pre.at[:T0 + 1].set(
        jnp.concatenate([cls, jnp.zeros((T0, D), jnp.float32)], axis=0) + pos)
    sel = jnp.zeros((TP, T0), jnp.float32)
    sel = sel.at[1 + jnp.arange(T0), jnp.arange(T0)].set(1.0)
    msk = jnp.where(jnp.arange(TP)[None, :] < T0 + 1, 0.0, -1e9
                    ).astype(jnp.float32)

    def cath(name):
        return jnp.stack(
            [jnp.transpose(b[name], (1, 0, 2)).reshape(D, heads * dh)
             for b in blocks], axis=0)

    wqkv = jnp.concatenate([cath("wq"), cath("wk"), cath("wv")],
                           axis=2).astype(bf)
    wo = jnp.stack([b["wo"].reshape(heads * dh, D) for b in blocks],
                   axis=0).astype(bf)
    w1 = jnp.stack([b["w1"] for b in blocks], 0).astype(bf)
    w2 = jnp.stack([b["w2"] for b in blocks], 0).astype(bf)
    stk1 = lambda n: jnp.stack([b[n] for b in blocks], 0)[:, None, :]
    b1, b2 = stk1("b1"), stk1("b2")
    l1g, l1b, l2g, l2b = (stk1("ln1_g"), stk1("ln1_b"),
                          stk1("ln2_g"), stk1("ln2_b"))

    weights = [proj_w.astype(bf), proj_b[None, :], pre, sel.astype(bf), msk,
               wqkv, wo, w1, b1, w2, b2, l1g, l1b, l2g, l2b]
    in_specs = [pl.BlockSpec((pl.Squeezed(), T0, D), lambda n: (n, 0, 0))]
    for wgt in weights:
        in_specs.append(
            pl.BlockSpec(wgt.shape, lambda n, _nd=wgt.ndim: (0,) * _nd))

    cost = pl.CostEstimate(
        flops=N * B * (8 * TP * D * D + 4 * heads * TP * TP * dh
                       + 4 * TP * D * mlp),
        transcendentals=N * B * (heads * TP * TP + TP * mlp),
        bytes_accessed=2 * N * T0 * D + 4 * N * T0 * D
        + sum(int(w.size) * w.dtype.itemsize for w in weights))

    return pl.pallas_call(
        _vit_kernel(B, heads, TP, T0, D, dh, 1.0 / math.sqrt(dh)),
        out_shape=jax.ShapeDtypeStruct((N, T0, D), jnp.float32),
        grid=(N,),
        in_specs=in_specs,
        out_specs=pl.BlockSpec((pl.Squeezed(), T0, D), lambda n: (n, 0, 0)),
        compiler_params=pltpu.CompilerParams(
            dimension_semantics=("parallel",)),
        cost_estimate=cost,
    )(tokens_bf, *weights)


def kernel(x, conv_layer__w, conv_layer__gamma, conv_layer__beta, conv_layer__mean, conv_layer__var, encoder1__w, encoder1__gamma, encoder1__beta, encoder1__mean, encoder1__var, encoder2__w, encoder2__gamma, encoder2__beta, encoder2__mean, encoder2__var, encoder3__w, encoder3__gamma, encoder3__beta, encoder3__mean, encoder3__var, post_transformer__w, post_transformer__gamma, post_transformer__beta, post_transformer__mean, post_transformer__var, post_transformer__b, vit__proj_w, vit__proj_b, vit__cls, vit__pos, vit_block0__wq, vit_block0__wk, vit_block0__wv, vit_block0__wo, vit_block0__ln1_g, vit_block0__ln1_b, vit_block0__w1, vit_block0__b1, vit_block0__w2, vit_block0__b2, vit_block0__ln2_g, vit_block0__ln2_b, vit_block1__wq, vit_block1__wk, vit_block1__wv, vit_block1__wo, vit_block1__ln1_g, vit_block1__ln1_b, vit_block1__w1, vit_block1__b1, vit_block1__w2, vit_block1__b2, vit_block1__ln2_g, vit_block1__ln2_b, vit_block2__wq, vit_block2__wk, vit_block2__wv, vit_block2__wo, vit_block2__ln1_g, vit_block2__ln1_b, vit_block2__w1, vit_block2__b1, vit_block2__w2, vit_block2__b2, vit_block2__ln2_g, vit_block2__ln2_b, vit_block3__wq, vit_block3__wk, vit_block3__wv, vit_block3__wo, vit_block3__ln1_g, vit_block3__ln1_b, vit_block3__w1, vit_block3__b1, vit_block3__w2, vit_block3__b2, vit_block3__ln2_g, vit_block3__ln2_b):
    h = jnp.transpose(x, (0, 2, 3, 1))

    x1 = _conv_s2_packed(h, conv_layer__w, conv_layer__gamma,
                         conv_layer__beta, conv_layer__mean, conv_layer__var,
                         k=7, pad=3)
    x2 = _conv_s2_packed(x1, encoder1__w, encoder1__gamma, encoder1__beta,
                         encoder1__mean, encoder1__var, k=3, pad=1)
    x3 = _conv_s2_packed(x2, encoder2__w, encoder2__gamma, encoder2__beta,
                         encoder2__mean, encoder2__var, k=3, pad=1)
    x4 = _conv_plain(x3, encoder3__w, encoder3__gamma, encoder3__beta,
                     encoder3__mean, encoder3__var, stride=2)

    N, Hv, Wv, D = x4.shape
    tokens = x4.reshape(N, Hv * Wv, D).astype(jnp.bfloat16)

    blocks = []
    for i, pfx in enumerate([
            (vit_block0__wq, vit_block0__wk, vit_block0__wv, vit_block0__wo,
             vit_block0__ln1_g, vit_block0__ln1_b, vit_block0__w1,
             vit_block0__b1, vit_block0__w2, vit_block0__b2,
             vit_block0__ln2_g, vit_block0__ln2_b),
            (vit_block1__wq, vit_block1__wk, vit_block1__wv, vit_block1__wo,
             vit_block1__ln1_g, vit_block1__ln1_b, vit_block1__w1,
             vit_block1__b1, vit_block1__w2, vit_block1__b2,
             vit_block1__ln2_g, vit_block1__ln2_b),
            (vit_block2__wq, vit_block2__wk, vit_block2__wv, vit_block2__wo,
             vit_block2__ln1_g, vit_block2__ln1_b, vit_block2__w1,
             vit_block2__b1, vit_block2__w2, vit_block2__b2,
             vit_block2__ln2_g, vit_block2__ln2_b),
            (vit_block3__wq, vit_block3__wk, vit_block3__wv, vit_block3__wo,
             vit_block3__ln1_g, vit_block3__ln1_b, vit_block3__w1,
             vit_block3__b1, vit_block3__w2, vit_block3__b2,
             vit_block3__ln2_g, vit_block3__ln2_b)]):
        wq, wk, wv, wo, l1g, l1b, w1, b1, w2, b2, l2g, l2b = pfx
        blocks.append({"wq": wq, "wk": wk, "wv": wv, "wo": wo,
                       "ln1_g": l1g, "ln1_b": l1b, "w1": w1, "b1": b1,
                       "w2": w2, "b2": b2, "ln2_g": l2g, "ln2_b": l2b})

    tv = _vit(tokens, vit__proj_w, vit__proj_b, vit__cls, vit__pos, blocks)
    xv = tv.reshape(N, Hv, Wv, D)

    xo = _conv_plain(xv, post_transformer__w, post_transformer__gamma,
                     post_transformer__beta, post_transformer__mean,
                     post_transformer__var, stride=1, b=post_transformer__b)

    nchw = lambda t: jnp.transpose(t, (0, 3, 1, 2))
    return nchw(xo), nchw(x1), nchw(x2), nchw(x3)
